# baseline (speedup 1.0000x reference)
import jax
import jax.numpy as jnp
from jax import lax
from jax.experimental import pallas as pl
from jax.experimental.pallas import tpu as pltpu
from jax.experimental.pallas import tpu_sc as plsc

B, C, H, W, D = 2, 2, 128, 128, 128
Hp, Wp, Dp = H + 2, W + 2, D + 2
HWD = H * W * D
N_PTS = B * HWD
N_TAB = B * Hp * Wp * Dp
MAX_OFF = Wp * Dp + Dp + 1
TAIL = MAX_OFF + 9

NC, NS, L = 2, 16, 16
NW = NC * NS
PTS_W = N_PTS // NW
P = 2048
CHUNKS = PTS_W // P
VSTEPS = P // L


def _floor_i32(x):
    t = x.astype(jnp.int32)
    return jnp.where(t.astype(jnp.float32) > x, t - 1, t)


def _warp_body(tab_hbm, dm_hbm, out_hbm,
               dxr, dyr, dzr, idxr, g2, o0r, o1r, sem, *wrs):
    wid = lax.axis_index("s") * NC + lax.axis_index("c")
    base_pt = wid * PTS_W
    b = base_pt // HWD
    dm_b = b * 3 * HWD
    out_b = b * C * HWD
    tab_b = b * (Hp * Wp * Dp)

    def chunk(t, _):
        local0 = (base_pt % HWD) + t * P
        pltpu.sync_copy(dm_hbm.at[pl.ds(dm_b + 0 * HWD + local0, P)], dxr)
        pltpu.sync_copy(dm_hbm.at[pl.ds(dm_b + 1 * HWD + local0, P)], dyr)
        pltpu.sync_copy(dm_hbm.at[pl.ds(dm_b + 2 * HWD + local0, P)], dzr)

        iota = lax.iota(jnp.int32, L)

        def compute(v, _):
            sl = pl.ds(v * L, L)
            g = local0 + v * L + iota
            hh = (g >> 14) & 127
            ww = (g >> 7) & 127
            dd = g & 127
            x = dxr[sl] + hh.astype(jnp.float32) + 1.0
            y = dyr[sl] + ww.astype(jnp.float32) + 1.0
            z = dzr[sl] + dd.astype(jnp.float32) + 1.0
            x0f = _floor_i32(x)
            y0f = _floor_i32(y)
            z0f = _floor_i32(z)
            x0 = jnp.clip(x0f, 0, Hp - 1)
            x1 = jnp.clip(x0f + 1, 0, Hp - 1)
            y0 = jnp.clip(y0f, 0, Wp - 1)
            y1 = jnp.clip(y0f + 1, 0, Wp - 1)
            z0 = jnp.clip(z0f, 0, Dp - 1)
            z1 = jnp.clip(z0f + 1, 0, Dp - 1)
            ddx = x1.astype(jnp.float32) - x
            ddy = y1.astype(jnp.float32) - y
            ddz = z1.astype(jnp.float32) - z
            ex, ey, ez = 1.0 - ddx, 1.0 - ddy, 1.0 - ddz
            idxr[sl] = tab_b + x0 * (Wp * Dp) + y0 * Dp + z0
            xy00 = ddx * ddy
            xy01 = ddx * ey
            xy10 = ex * ddy
            xy11 = ex * ey
            w = [xy00 * ddz, xy00 * ez, xy01 * ddz, xy01 * ez,
                 xy10 * ddz, xy10 * ez, xy11 * ddz, xy11 * ez]
            dup_x = x1 == x0
            dup_y = y1 == y0
            dup_z = z1 == z0
            zf = jnp.zeros((L,), jnp.float32)
            for k in range(4):
                w[k] = jnp.where(dup_x, w[k] + w[k + 4], w[k])
                w[k + 4] = jnp.where(dup_x, zf, w[k + 4])
            for k in (0, 1, 4, 5):
                w[k] = jnp.where(dup_y, w[k] + w[k + 2], w[k])
                w[k + 2] = jnp.where(dup_y, zf, w[k + 2])
            for k in (0, 2, 4, 6):
                w[k] = jnp.where(dup_z, w[k] + w[k + 1], w[k])
                w[k + 1] = jnp.where(dup_z, zf, w[k + 1])
            for j in range(8):
                wrs[j][sl] = w[j]
            return ()

        lax.fori_loop(0, VSTEPS, compute, (), unroll=False)

        pltpu.async_copy(tab_hbm.at[idxr], g2, sem).wait()

        def combine(v, _):
            sl = pl.ds(v * L, L)
            rows = v * L + iota
            acc0 = jnp.zeros((L,), jnp.float32)
            acc1 = jnp.zeros((L,), jnp.float32)
            for j in range(8):
                wj = wrs[j][sl]
                acc0 = acc0 + wj * plsc.load_gather(
                    g2, [rows, jnp.full((L,), 2 * j, jnp.int32)])
                acc1 = acc1 + wj * plsc.load_gather(
                    g2, [rows, jnp.full((L,), 2 * j + 1, jnp.int32)])
            o0r[sl] = acc0
            o1r[sl] = acc1
            return ()

        lax.fori_loop(0, VSTEPS, combine, (), unroll=False)

        pltpu.sync_copy(o0r, out_hbm.at[pl.ds(out_b + 0 * HWD + local0, P)])
        pltpu.sync_copy(o1r, out_hbm.at[pl.ds(out_b + 1 * HWD + local0, P)])
        return ()

    lax.fori_loop(0, CHUNKS, chunk, (), unroll=False)


@jax.jit
def _warp(tab, dm_flat):
    mesh = plsc.VectorSubcoreMesh(core_axis_name="c", subcore_axis_name="s",
                                  num_cores=NC, num_subcores=NS)
    f = pl.kernel(
        _warp_body,
        out_type=jax.ShapeDtypeStruct((B * C * HWD,), jnp.float32),
        mesh=mesh,
        compiler_params=pltpu.CompilerParams(use_tc_tiling_on_sc=False,
                                             needs_layout_passes=False),
        scratch_types=[
            pltpu.VMEM((P,), jnp.float32),
            pltpu.VMEM((P,), jnp.float32),
            pltpu.VMEM((P,), jnp.float32),
            pltpu.VMEM((P,), jnp.int32),
            pltpu.VMEM((P, 16), jnp.float32),
            pltpu.VMEM((P,), jnp.float32),
            pltpu.VMEM((P,), jnp.float32),
            pltpu.SemaphoreType.DMA,
        ]
        + [pltpu.VMEM((P,), jnp.float32) for _ in range(8)],
    )
    return f(tab, dm_flat)


def kernel(moving_image, deformation_matrix):
    assert moving_image.shape == (B, C, H, W, D)
    assert deformation_matrix.shape == (B, 3, H, W, D)
    im = jnp.pad(moving_image, ((0, 0), (0, 0), (1, 1), (1, 1), (1, 1)))
    ext = jnp.concatenate(
        [im.transpose(1, 0, 2, 3, 4).reshape(C, -1),
         jnp.zeros((C, TAIL), jnp.float32)], axis=1)
    cols = []
    for dx_ in (0, 1):
        for dy_ in (0, 1):
            for dz_ in (0, 1):
                off = dx_ * (Wp * Dp) + dy_ * Dp + dz_
                for c in range(C):
                    cols.append(lax.slice(ext[c], (off,), (off + N_TAB,)))
    tab = jnp.stack(cols, axis=1)
    dm_flat = deformation_matrix.reshape(-1)
    out = _warp(tab, dm_flat)
    return out.reshape(B, C, H, W, D)


# speedup vs baseline: 10.1790x; 10.1790x over previous
import jax
import jax.numpy as jnp
from jax import lax
from jax.experimental import pallas as pl
from jax.experimental.pallas import tpu as pltpu
from jax.experimental.pallas import tpu_sc as plsc

B, C, H, W, D = 2, 2, 128, 128, 128
Hp, Wp, Dp = H + 2, W + 2, D + 2
HWD = H * W * D
N_PTS = B * HWD
N_TAB = B * Hp * Wp * Dp
TROW = 17
MAX_OFF = Wp * Dp + Dp + 1

NC, NS, L = 2, 16, 16
NW = NC * NS

PLANES = B * Hp
PLANE_ROWS = Wp * Dp
YC = 5
YCH = Wp // YC
PLANE_ITERS = (PLANES + NW - 1) // NW

PTS_W = N_PTS // NW
P = 1024
CHUNKS = PTS_W // P
VSTEPS = P // L

_SC_PARAMS = pltpu.CompilerParams(use_tc_tiling_on_sc=False,
                                  needs_layout_passes=False)
_MESH = dict(core_axis_name="c", subcore_axis_name="s",
             num_cores=NC, num_subcores=NS)

def _floor_i32(x):
    t = x.astype(jnp.int32)
    return jnp.where(t.astype(jnp.float32) > x, t - 1, t)


def _build_body(im_hbm, tab_hbm, tlocs, p00, p01, p10, p11, sem, osems):
    wid = lax.axis_index("s") * NC + lax.axis_index("c")

    iota = lax.iota(jnp.int32, L)
    iota16 = iota * 16
    planes = (p00, p01, p10, p11)

    ZB = [16 * k for k in range(8)] + [Dp - L]
    zvm = []
    for k in range(9):
        z = ZB[k] + iota
        zvm.append([((z - 1) >= 0) & ((z - 1) <= D - 1),
                    (z >= 0) & (z <= D - 1)])

    def plane_step(t, _):
        q = wid + NW * t

        @pl.when(q < PLANES)
        def _():
            b = q // Hp
            x = q % Hp
            xc0 = jnp.clip(x - 1, 0, H - 1)
            xc1 = jnp.clip(x, 0, H - 1)
            copies = []
            for dx_, xs in ((0, xc0), (1, xc1)):
                for c in range(C):
                    off = ((b * C + c) * H + xs) * (W * D)
                    copies.append(pltpu.async_copy(
                        im_hbm.at[pl.ds(off, W * D)],
                        planes[dx_ * C + c].at[pl.ds(8, W * D)], sem))
            for cp in copies:
                cp.wait()
            ones = jnp.ones((L,), jnp.int32)
            xv = (((x * ones) >= 1) & ((x * ones) <= H),
                  (x * ones) <= H - 1)
            zf = jnp.zeros((L,), jnp.float32)

            def ychunk(yc, par):
                tloc = tlocs[par]

                def yrow(yr, _):
                    y = yc * YC + yr
                    yvx = []
                    ybase = []
                    for dy_ in (0, 1):
                        ys = y + dy_ - 1
                        yv = ((ys * ones) >= 0) & ((ys * ones) <= W - 1)
                        yvx.append((yv & xv[0], yv & xv[1]))
                        ybase.append(jnp.clip(ys, 0, W - 1) * D + 8)
                    for k in range(9):
                        strow = yr * Dp + ZB[k] + iota
                        vals = {}
                        for dy_ in (0, 1):
                            for dz_ in (0, 1):
                                start = ybase[dy_] + ZB[k] + dz_ - 1
                                for dx_ in (0, 1):
                                    for c in range(C):
                                        vals[(dx_, dy_, dz_, c)] = \
                                            planes[dx_ * C + c][
                                                pl.ds(start, L)]
                        for col in range(16):
                            dx_ = col >> 3
                            dy_ = (col >> 2) & 1
                            dz_ = (col >> 1) & 1
                            c = col & 1
                            m = yvx[dy_][dx_] & zvm[k][dz_]
                            plsc.store_scatter(
                                tloc, [strow, jnp.full((L,), col, jnp.int32)],
                                jnp.where(m, vals[(dx_, dy_, dz_, c)], zf))
                    return ()

                lax.fori_loop(0, YC, yrow, (), unroll=False)
                return pltpu.async_copy(
                    tloc,
                    tab_hbm.at[pl.ds(q * PLANE_ROWS + yc * YC * Dp,
                                     YC * Dp)],
                    osems[par])

            def ycpair(g, _):
                for par in (0, 1):
                    yc = 2 * g + par

                    @pl.when(g > 0)
                    def _():
                        pltpu.make_async_copy(
                            tlocs[par],
                            tab_hbm.at[pl.ds(
                                q * PLANE_ROWS + (yc - 2) * YC * Dp,
                                YC * Dp)],
                            osems[par]).wait()
                    ychunk(yc, par)
                return ()

            lax.fori_loop(0, YCH // 2, ycpair, (), unroll=False)
            for par in (0, 1):
                pltpu.make_async_copy(
                    tlocs[par],
                    tab_hbm.at[pl.ds(
                        q * PLANE_ROWS + (YCH - 2 + par) * YC * Dp,
                        YC * Dp)],
                    osems[par]).wait()

        return ()

    lax.fori_loop(0, PLANE_ITERS, plane_step, (), unroll=False)


@jax.jit
def _build_table(im_flat):
    f = pl.kernel(
        _build_body,
        out_type=jax.ShapeDtypeStruct((N_TAB, TROW), jnp.float32),
        mesh=plsc.VectorSubcoreMesh(**_MESH),
        compiler_params=_SC_PARAMS,
        name="sc_build",
        scratch_types=[
            [pltpu.VMEM((YC * Dp, TROW), jnp.float32) for _ in range(2)],
            pltpu.VMEM((W * D + 16,), jnp.float32),
            pltpu.VMEM((W * D + 16,), jnp.float32),
            pltpu.VMEM((W * D + 16,), jnp.float32),
            pltpu.VMEM((W * D + 16,), jnp.float32),
            pltpu.SemaphoreType.DMA,
            [pltpu.SemaphoreType.DMA, pltpu.SemaphoreType.DMA],
        ],
    )
    return f(im_flat)


def _warp_body(tab_hbm, dm_hbm, out_hbm,
               dms, idxs, g2s, gsems, dmsems, o0r, o1r, ws):
    wid = lax.axis_index("s") * NC + lax.axis_index("c")
    base_pt = wid * PTS_W
    b = base_pt // HWD
    dm_b = b * 3 * HWD
    out_b = b * C * HWD
    tab_b = b * (Hp * Wp * Dp)
    local_w = base_pt % HWD
    iota = lax.iota(jnp.int32, L)

    def dm_copies(t, par):
        local0 = local_w + t * P
        return [pltpu.make_async_copy(
            dm_hbm.at[pl.ds(dm_b + k * HWD + local0, P)], dms[par][k],
            dmsems[par]) for k in range(3)]

    def compute(t, par):
        dxr, dyr, dzr = dms[par]
        local0 = local_w + t * P
        wrs = ws[par]

        def step(v, _):
            sl = pl.ds(v * L, L)
            g = local0 + v * L + iota
            hh = (g >> 14) & 127
            ww = (g >> 7) & 127
            dd = g & 127
            x = dxr[sl] + hh.astype(jnp.float32) + 1.0
            y = dyr[sl] + ww.astype(jnp.float32) + 1.0
            z = dzr[sl] + dd.astype(jnp.float32) + 1.0
            x0f = _floor_i32(x)
            y0f = _floor_i32(y)
            z0f = _floor_i32(z)
            x0 = jnp.clip(x0f, 0, Hp - 1)
            x1 = jnp.clip(x0f + 1, 0, Hp - 1)
            y0 = jnp.clip(y0f, 0, Wp - 1)
            y1 = jnp.clip(y0f + 1, 0, Wp - 1)
            z0 = jnp.clip(z0f, 0, Dp - 1)
            z1 = jnp.clip(z0f + 1, 0, Dp - 1)
            ddx = x1.astype(jnp.float32) - x
            ddy = y1.astype(jnp.float32) - y
            ddz = z1.astype(jnp.float32) - z
            ex, ey, ez = 1.0 - ddx, 1.0 - ddy, 1.0 - ddz
            idxs[par][sl] = tab_b + x0 * (Wp * Dp) + y0 * Dp + z0
            xy00 = ddx * ddy
            xy01 = ddx * ey
            xy10 = ex * ddy
            xy11 = ex * ey
            w = [xy00 * ddz, xy00 * ez, xy01 * ddz, xy01 * ez,
                 xy10 * ddz, xy10 * ez, xy11 * ddz, xy11 * ez]
            dup_x = x1 == x0
            dup_y = y1 == y0
            dup_z = z1 == z0
            zf = jnp.zeros((L,), jnp.float32)
            for k in range(4):
                w[k] = jnp.where(dup_x, w[k] + w[k + 4], w[k])
                w[k + 4] = jnp.where(dup_x, zf, w[k + 4])
            for k in (0, 1, 4, 5):
                w[k] = jnp.where(dup_y, w[k] + w[k + 2], w[k])
                w[k + 2] = jnp.where(dup_y, zf, w[k + 2])
            for k in (0, 2, 4, 6):
                w[k] = jnp.where(dup_z, w[k] + w[k + 1], w[k])
                w[k + 1] = jnp.where(dup_z, zf, w[k + 1])
            for j in range(8):
                wrs[j][sl] = w[j]
            return ()

        lax.fori_loop(0, VSTEPS, step, (), unroll=2)

    def gather_copy(par):
        return pltpu.make_async_copy(tab_hbm.at[idxs[par]], g2s[par],
                                     gsems[par])

    def combine(t, par):
        wrs = ws[par]
        g2 = g2s[par]
        local0 = local_w + t * P

        def step(v, _):
            sl = pl.ds(v * L, L)
            rows = v * L + iota
            wv = [wrs[j][sl] for j in range(8)]
            p0 = [wv[j] * plsc.load_gather(
                g2, [rows, jnp.full((L,), 2 * j, jnp.int32)])
                for j in range(8)]
            p1 = [wv[j] * plsc.load_gather(
                g2, [rows, jnp.full((L,), 2 * j + 1, jnp.int32)])
                for j in range(8)]
            o0r[sl] = (((p0[0] + p0[1]) + (p0[2] + p0[3]))
                       + ((p0[4] + p0[5]) + (p0[6] + p0[7])))
            o1r[sl] = (((p1[0] + p1[1]) + (p1[2] + p1[3]))
                       + ((p1[4] + p1[5]) + (p1[6] + p1[7])))
            return ()

        lax.fori_loop(0, VSTEPS, step, (), unroll=2)
        pltpu.sync_copy(o0r, out_hbm.at[pl.ds(out_b + 0 * HWD + local0, P)])
        pltpu.sync_copy(o1r, out_hbm.at[pl.ds(out_b + 1 * HWD + local0, P)])

    for cp in dm_copies(0, 0):
        cp.start()

    def pair(gi, _):
        for par in (0, 1):
            t = 2 * gi + par
            for cp in dm_copies(t, par):
                cp.wait()

            @pl.when(t + 1 < CHUNKS)
            def _():
                for cp in dm_copies(t + 1, 1 - par):
                    cp.start()

            with jax.named_scope("wcompute"):
                compute(t, par)
            gather_copy(par).start()

            @pl.when(t > 0)
            def _():
                with jax.named_scope("wgwait"):
                    gather_copy(1 - par).wait()
                with jax.named_scope("wcombine"):
                    combine(t - 1, 1 - par)
        return ()

    lax.fori_loop(0, CHUNKS // 2, pair, (), unroll=False)
    gather_copy(1).wait()
    combine(CHUNKS - 1, 1)


@jax.jit
def _warp(tab, dm_flat):
    dbl = lambda shape, dt: [pltpu.VMEM(shape, dt) for _ in range(2)]
    f = pl.kernel(
        _warp_body,
        out_type=jax.ShapeDtypeStruct((B * C * HWD,), jnp.float32),
        mesh=plsc.VectorSubcoreMesh(**_MESH),
        compiler_params=_SC_PARAMS,
        name="sc_warp",
        scratch_types=[
            [[pltpu.VMEM((P,), jnp.float32) for _ in range(3)]
             for _ in range(2)],
            dbl((P,), jnp.int32),
            dbl((P, TROW), jnp.float32),
            [pltpu.SemaphoreType.DMA, pltpu.SemaphoreType.DMA],
            [pltpu.SemaphoreType.DMA, pltpu.SemaphoreType.DMA],
            pltpu.VMEM((P,), jnp.float32),
            pltpu.VMEM((P,), jnp.float32),
            [[pltpu.VMEM((P,), jnp.float32) for _ in range(8)]
             for _ in range(2)],
        ],
    )
    return f(tab, dm_flat)


def kernel(moving_image, deformation_matrix):
    assert moving_image.shape == (B, C, H, W, D)
    assert deformation_matrix.shape == (B, 3, H, W, D)
    tab = _build_table(moving_image.reshape(-1))
    dm_flat = deformation_matrix.reshape(-1)
    out = _warp(tab, dm_flat)
    return out.reshape(B, C, H, W, D)
